# baseline (speedup 1.0000x reference)
import functools

import jax
import jax.numpy as jnp
from jax import lax
from jax.experimental import pallas as pl
from jax.experimental.pallas import tpu as pltpu
from jax.experimental.pallas import tpu_sc as plsc

_N = 10000
_E = 320000
_DIN = 128
_HID = 72
_HP = 80
_NL = 3

_NC = 2
_NS = 16
_NW = _NC * _NS
_ET = _E // _NW
_CH = 80
_NCH = _ET // _CH
_RB = 1000
_NRB = _N // _RB
_HP2 = 128
_BE = 3200
_NBE = _E // _BE
_ZR = 200


def _sc_mesh():
    return plsc.VectorSubcoreMesh(core_axis_name="c", subcore_axis_name="s")


def _worker(base_len):
    cid = lax.axis_index("c")
    sid = lax.axis_index("s")
    wid = sid * _NC + cid
    return cid, sid, wid * base_len


def _sc_deg_body(col2_hbm, out0_hbm, out1_hbm, idx_c2, ones_v, zb_v, acc_sh, sem):
    cid, sid, _ = _worker(0)
    wid = sid * _NC + cid

    def fill_ones(k, carry):
        ones_v[pl.ds(k * 16, 16)] = jnp.full((16,), 1.0, jnp.float32)
        return carry
    lax.fori_loop(0, _CH // 16, fill_ones, 0)

    def fill_z(k, carry):
        zb_v[pl.ds(k * 16, 16)] = jnp.zeros((16,), jnp.float32)
        return carry
    lax.fori_loop(0, 1024 // 16, fill_z, 0)

    @pl.when(sid < _NRB)
    def _():
        pltpu.sync_copy(zb_v.at[pl.ds(0, _RB)], acc_sh.at[pl.ds(sid * _RB, _RB)])
    plsc.subcore_barrier()

    pltpu.sync_copy(col2_hbm.at[wid], idx_c2)

    def step(j, carry):
        pltpu.sync_copy(ones_v, acc_sh.at[idx_c2.at[j]], add=True)
        return carry
    lax.fori_loop(0, _NCH, step, 0)
    plsc.subcore_barrier()

    @pl.when(sid < _NRB)
    def _():
        pltpu.sync_copy(acc_sh.at[pl.ds(sid * _RB, _RB)], zb_v.at[pl.ds(0, _RB)])

    @pl.when((sid < _NRB) & (cid == 0))
    def _():
        pltpu.sync_copy(zb_v.at[pl.ds(0, _RB)], out0_hbm.at[pl.ds(sid * _RB, _RB)])

    @pl.when((sid < _NRB) & (cid == 1))
    def _():
        pltpu.sync_copy(zb_v.at[pl.ds(0, _RB)], out1_hbm.at[pl.ds(sid * _RB, _RB)])


def _sc_deg(col2):
    f = pl.kernel(
        _sc_deg_body,
        out_type=(jax.ShapeDtypeStruct((_N,), jnp.float32),
                  jax.ShapeDtypeStruct((_N,), jnp.float32)),
        mesh=_sc_mesh(),
        compiler_params=pltpu.CompilerParams(use_tc_tiling_on_sc=False),
        scratch_types=[
            pltpu.VMEM((_NCH, _CH), jnp.int32),
            pltpu.VMEM((_CH,), jnp.float32),
            pltpu.VMEM((1024,), jnp.float32),
            pltpu.VMEM_SHARED((_N,), jnp.float32),
            pltpu.SemaphoreType.DMA,
        ],
    )
    return f(col2)


def _sc_spmm_body(zt_hbm, row2_hbm, col2_hbm, out_hbm,
                  idx_r2, idx_c2, rows_a, rows_b, zb_v, acc_sh, sem_a, sem_b):
    cid, sid, _ = _worker(0)
    wid = sid * _NC + cid

    def fill_z(k, carry):
        i = k // (_HP // 16)
        j = k % (_HP // 16)
        zb_v[i, pl.ds(j * 16, 16)] = jnp.zeros((16,), jnp.float32)
        return carry
    lax.fori_loop(0, _ZR * (_HP // 16), fill_z, 0)

    @pl.when(sid < _NRB)
    def _():
        def zcp(t, carry):
            pltpu.sync_copy(zb_v, acc_sh.at[pl.ds(sid * _RB + t * _ZR, _ZR)])
            return carry
        lax.fori_loop(0, _RB // _ZR, zcp, 0)
    plsc.subcore_barrier()

    pltpu.sync_copy(row2_hbm.at[wid], idx_r2)
    pltpu.sync_copy(col2_hbm.at[wid], idx_c2)

    def start(j, rv, s):
        pltpu.async_copy(zt_hbm.at[idx_r2.at[j]], rv, s)

    def finish(j, rv, s):
        pltpu.make_async_copy(zt_hbm.at[idx_r2.at[j]], rv, s).wait()
        pltpu.sync_copy(rv, acc_sh.at[idx_c2.at[j]], add=True)

    start(0, rows_a, sem_a)

    def step(t, carry):
        start(2 * t + 1, rows_b, sem_b)
        finish(2 * t, rows_a, sem_a)
        start(2 * t + 2, rows_a, sem_a)
        finish(2 * t + 1, rows_b, sem_b)
        return carry
    lax.fori_loop(0, (_NCH - 1) // 2, step, 0)
    finish(_NCH - 1, rows_a, sem_a)
    plsc.subcore_barrier()

    @pl.when(sid < _NRB)
    def _():
        def wcp(t, carry):
            off = sid * _RB + t * _ZR
            pltpu.sync_copy(acc_sh.at[pl.ds(off, _ZR)], zb_v)
            pltpu.sync_copy(zb_v, out_hbm.at[cid, pl.ds(off, _ZR)])
            return carry
        lax.fori_loop(0, _RB // _ZR, wcp, 0)


def _sc_spmm(zt, row2, col2):
    f = pl.kernel(
        _sc_spmm_body,
        out_type=jax.ShapeDtypeStruct((_NC, _N, _HP), jnp.float32),
        mesh=_sc_mesh(),
        compiler_params=pltpu.CompilerParams(use_tc_tiling_on_sc=False),
        scratch_types=[
            pltpu.VMEM((_NCH, _CH), jnp.int32),
            pltpu.VMEM((_NCH, _CH), jnp.int32),
            pltpu.VMEM((_CH, _HP), jnp.float32),
            pltpu.VMEM((_CH, _HP), jnp.float32),
            pltpu.VMEM((_ZR, _HP), jnp.float32),
            pltpu.VMEM_SHARED((_N, _HP), jnp.float32),
            pltpu.SemaphoreType.DMA,
            pltpu.SemaphoreType.DMA,
        ],
    )
    return f(zt, row2, col2)


def _sc_gather2_body(p_hbm, q_hbm, row2_hbm, col2_hbm, pe_hbm, qe_hbm,
                     idx_r2, idx_c2, pr_a, qr_a, pr_b, qr_b,
                     sp_a, sq_a, sp_b, sq_b):
    cid, sid, ebase = _worker(_ET)
    wid = sid * _NC + cid

    pltpu.sync_copy(row2_hbm.at[wid], idx_r2)
    pltpu.sync_copy(col2_hbm.at[wid], idx_c2)

    def start(j, pr, qr, sp, sq):
        pltpu.async_copy(p_hbm.at[idx_r2.at[j]], pr, sp)
        pltpu.async_copy(q_hbm.at[idx_c2.at[j]], qr, sq)

    def finish(j, pr, qr, sp, sq):
        pltpu.make_async_copy(p_hbm.at[idx_r2.at[j]], pr, sp).wait()
        pltpu.make_async_copy(q_hbm.at[idx_c2.at[j]], qr, sq).wait()
        off = ebase + j * _CH
        pltpu.sync_copy(pr, pe_hbm.at[pl.ds(off, _CH)])
        pltpu.sync_copy(qr, qe_hbm.at[pl.ds(off, _CH)])

    start(0, pr_a, qr_a, sp_a, sq_a)

    def step(t, carry):
        start(2 * t + 1, pr_b, qr_b, sp_b, sq_b)
        finish(2 * t, pr_a, qr_a, sp_a, sq_a)
        start(2 * t + 2, pr_a, qr_a, sp_a, sq_a)
        finish(2 * t + 1, pr_b, qr_b, sp_b, sq_b)
        return carry
    lax.fori_loop(0, (_NCH - 1) // 2, step, 0)
    finish(_NCH - 1, pr_a, qr_a, sp_a, sq_a)


def _sc_gather2(p, q, row2, col2):
    f = pl.kernel(
        _sc_gather2_body,
        out_type=(jax.ShapeDtypeStruct((_E, _HP2), jnp.float32),
                  jax.ShapeDtypeStruct((_E, _HP2), jnp.float32)),
        mesh=_sc_mesh(),
        compiler_params=pltpu.CompilerParams(use_tc_tiling_on_sc=False),
        scratch_types=[
            pltpu.VMEM((_NCH, _CH), jnp.int32),
            pltpu.VMEM((_NCH, _CH), jnp.int32),
            pltpu.VMEM((_CH, _HP2), jnp.float32),
            pltpu.VMEM((_CH, _HP2), jnp.float32),
            pltpu.VMEM((_CH, _HP2), jnp.float32),
            pltpu.VMEM((_CH, _HP2), jnp.float32),
            pltpu.SemaphoreType.DMA,
            pltpu.SemaphoreType.DMA,
            pltpu.SemaphoreType.DMA,
            pltpu.SemaphoreType.DMA,
        ],
    )
    return f(p, q, row2, col2)


def _dinv_of(deg):
    return jnp.where(deg > 0, lax.rsqrt(jnp.maximum(deg, 1e-12)), 0.0)


def _tc_node_body(x_ref, w_ref, b_ref, o_ref):
    o_ref[...] = jax.nn.relu(
        jnp.dot(x_ref[...], w_ref[...], preferred_element_type=jnp.float32)
        + b_ref[...])


def _tc_node(x, w, b):
    return pl.pallas_call(
        _tc_node_body,
        grid=(_NRB,),
        in_specs=[pl.BlockSpec((_RB, _DIN), lambda i: (i, 0)),
                  pl.BlockSpec((_DIN, _HP), lambda i: (0, 0)),
                  pl.BlockSpec((1, _HP), lambda i: (0, 0))],
        out_specs=pl.BlockSpec((_RB, _HP), lambda i: (i, 0)),
        out_shape=jax.ShapeDtypeStruct((_N, _HP), jnp.float32),
    )(x, w, b)


def _tc_pre_body(src_ref, k1_ref, k2_ref, deg_ref, w_ref, o_ref):
    h = src_ref[...] * k1_ref[...] + k2_ref[...]
    dinv = _dinv_of(deg_ref[...])
    o_ref[...] = jnp.dot(h, w_ref[...],
                         preferred_element_type=jnp.float32) * dinv


def _tc_pre(src, k1, k2, deg, w):
    return pl.pallas_call(
        _tc_pre_body,
        grid=(_NRB,),
        in_specs=[pl.BlockSpec((_RB, _HP), lambda i: (i, 0)),
                  pl.BlockSpec((1, _HP), lambda i: (0, 0)),
                  pl.BlockSpec((1, _HP), lambda i: (0, 0)),
                  pl.BlockSpec((_RB, 1), lambda i: (i, 0)),
                  pl.BlockSpec((_HP, _HP), lambda i: (0, 0))],
        out_specs=pl.BlockSpec((_RB, _HP), lambda i: (i, 0)),
        out_shape=jax.ShapeDtypeStruct((_N, _HP), jnp.float32),
    )(src, k1, k2, deg, w)


def _tc_post_body(p0_ref, p1_ref, src_ref, k1_ref, k2_ref, deg_ref, w_ref,
                  b_ref, o_ref, s1_ref, s2_ref):
    h = src_ref[...] * k1_ref[...] + k2_ref[...]
    dinv = _dinv_of(deg_ref[...])
    agg = (p0_ref[...] + p1_ref[...]) * dinv
    o = jax.nn.relu(agg
                    + jnp.dot(h, w_ref[...], preferred_element_type=jnp.float32)
                    + b_ref[...])
    o_ref[...] = o
    s1_ref[...] = jnp.sum(o, axis=0, keepdims=True)[None]
    s2_ref[...] = jnp.sum(o * o, axis=0, keepdims=True)[None]


def _tc_post(p0, p1, src, k1, k2, deg, w, b):
    return pl.pallas_call(
        _tc_post_body,
        grid=(_NRB,),
        in_specs=[pl.BlockSpec((_RB, _HP), lambda i: (i, 0)),
                  pl.BlockSpec((_RB, _HP), lambda i: (i, 0)),
                  pl.BlockSpec((_RB, _HP), lambda i: (i, 0)),
                  pl.BlockSpec((1, _HP), lambda i: (0, 0)),
                  pl.BlockSpec((1, _HP), lambda i: (0, 0)),
                  pl.BlockSpec((_RB, 1), lambda i: (i, 0)),
                  pl.BlockSpec((_HP, _HP), lambda i: (0, 0)),
                  pl.BlockSpec((1, _HP), lambda i: (0, 0))],
        out_specs=[pl.BlockSpec((_RB, _HP), lambda i: (i, 0)),
                   pl.BlockSpec((1, 1, _HP), lambda i: (i, 0, 0)),
                   pl.BlockSpec((1, 1, _HP), lambda i: (i, 0, 0))],
        out_shape=[jax.ShapeDtypeStruct((_N, _HP), jnp.float32),
                   jax.ShapeDtypeStruct((_NRB, 1, _HP), jnp.float32),
                   jax.ShapeDtypeStruct((_NRB, 1, _HP), jnp.float32)],
    )(p0, p1, src, k1, k2, deg, w, b)


def _tc_pq_body(src_ref, k1_ref, k2_ref, wa_ref, wb_ref, b1_ref,
                p_ref, q_ref):
    h = src_ref[...] * k1_ref[...] + k2_ref[...]
    p_ref[...] = jnp.dot(h, wa_ref[...],
                         preferred_element_type=jnp.float32) + b1_ref[...]
    q_ref[...] = jnp.dot(h, wb_ref[...], preferred_element_type=jnp.float32)


def _tc_pq(src, k1, k2, wa, wb, b1):
    return pl.pallas_call(
        _tc_pq_body,
        grid=(_NRB,),
        in_specs=[pl.BlockSpec((_RB, _HP), lambda i: (i, 0)),
                  pl.BlockSpec((1, _HP), lambda i: (0, 0)),
                  pl.BlockSpec((1, _HP), lambda i: (0, 0)),
                  pl.BlockSpec((_HP, _HP2), lambda i: (0, 0)),
                  pl.BlockSpec((_HP, _HP2), lambda i: (0, 0)),
                  pl.BlockSpec((1, _HP2), lambda i: (0, 0))],
        out_specs=[pl.BlockSpec((_RB, _HP2), lambda i: (i, 0)),
                   pl.BlockSpec((_RB, _HP2), lambda i: (i, 0))],
        out_shape=[jax.ShapeDtypeStruct((_N, _HP2), jnp.float32),
                   jax.ShapeDtypeStruct((_N, _HP2), jnp.float32)],
    )(src, k1, k2, wa, wb, b1)


def _tc_edge_body(pe_ref, qe_ref, ea_ref, v1_ref, b21_ref, w1c_ref, w2c_ref,
                  cv_ref, w3_ref, b3_ref, o_ref):
    t1 = jnp.tanh(pe_ref[...] + qe_ref[...])
    t2 = jnp.tanh(jnp.dot(ea_ref[...], v1_ref[...],
                          preferred_element_type=jnp.float32) + b21_ref[...])
    s = jnp.tanh(jnp.dot(t1.astype(jnp.bfloat16), w1c_ref[...],
                         preferred_element_type=jnp.float32)
                 + jnp.dot(t2.astype(jnp.bfloat16), w2c_ref[...],
                           preferred_element_type=jnp.float32)
                 + cv_ref[...])
    o = lax.dot_general(w3_ref[...], s, (((1,), (1,)), ((), ())),
                        preferred_element_type=jnp.float32) + b3_ref[...]
    o_ref[...] = o[None]


def _tc_edge(pe, qe, ea, v1, b21, w1c, w2c, cv, w3, b3):
    return pl.pallas_call(
        _tc_edge_body,
        grid=(_NBE,),
        in_specs=[pl.BlockSpec((_BE, _HP2), lambda i: (i, 0)),
                  pl.BlockSpec((_BE, _HP2), lambda i: (i, 0)),
                  pl.BlockSpec((_BE, 16), lambda i: (i, 0)),
                  pl.BlockSpec((16, _HP2), lambda i: (0, 0)),
                  pl.BlockSpec((1, _HP2), lambda i: (0, 0)),
                  pl.BlockSpec((_HP2, _HP2), lambda i: (0, 0)),
                  pl.BlockSpec((_HP2, _HP2), lambda i: (0, 0)),
                  pl.BlockSpec((1, _HP2), lambda i: (0, 0)),
                  pl.BlockSpec((1, _HP2), lambda i: (0, 0)),
                  pl.BlockSpec((1, 1), lambda i: (0, 0))],
        out_specs=pl.BlockSpec((1, 1, _BE), lambda i: (i, 0, 0)),
        out_shape=jax.ShapeDtypeStruct((_NBE, 1, _BE), jnp.float32),
    )(pe, qe, ea, v1, b21, w1c, w2c, cv, w3, b3)


def _padw(w, rows=_HP, cols=_HP):
    r, c = w.shape
    return jnp.pad(w, ((0, (rows - r) if r == _HID else 0), (0, cols - c)))


def _padv(v, cols=_HP):
    return jnp.pad(v.reshape(1, -1), ((0, 0), (0, cols - v.shape[-1])))


def kernel(x, edge_index, edge_attr, node_w, node_b, init_w, root_w, arma_b,
           bn_gamma, bn_beta, mlp1_w1, mlp1_b1, mlp1_w2, mlp1_b2,
           mlp2_w1, mlp2_b1, mlp2_w2, mlp2_b2,
           mlp3_w1, mlp3_b1, mlp3_w2, mlp3_b2):
    row2 = edge_index[0].reshape(_NW, _NCH, _CH)
    col2 = edge_index[1].reshape(_NW, _NCH, _CH)

    degp0, degp1 = _sc_deg(col2)
    deg = (degp0 + degp1).reshape(_N, 1)

    h = _tc_node(x, jnp.pad(node_w, ((0, 0), (0, _HP - _HID))), _padv(node_b))

    ones = jnp.ones((1, _HP), jnp.float32)
    zeros = jnp.zeros((1, _HP), jnp.float32)
    k1, k2, src = ones, zeros, h
    for l in range(_NL):
        zt = _tc_pre(src, k1, k2, deg, _padw(init_w[l]))
        part = _sc_spmm(zt, row2, col2)
        out, s1, s2 = _tc_post(part[0], part[1], src, k1, k2, deg,
                               _padw(root_w[l]), _padv(arma_b[l]))
        mean = jnp.sum(s1, axis=(0, 1)) * (1.0 / _N)
        var = jnp.sum(s2, axis=(0, 1)) * (1.0 / _N) - mean * mean
        g = _padv(bn_gamma[l])[0]
        k1v = g * lax.rsqrt(var + 1e-5)
        k2v = _padv(bn_beta[l])[0] - mean * k1v
        k1, k2, src = k1v.reshape(1, _HP), k2v.reshape(1, _HP), out

    wa = _padw(mlp1_w1[:_HID], cols=_HP2)
    wb = _padw(mlp1_w1[_HID:], cols=_HP2)
    p, q = _tc_pq(src, k1, k2, wa, wb, _padv(mlp1_b1, cols=_HP2))

    pe, qe = _sc_gather2(p, q, row2, col2)

    a3 = mlp3_w1[:_HID]
    b3 = mlp3_w1[_HID:]
    w1c = _padw(mlp1_w2 @ a3, rows=_HP2, cols=_HP2).astype(jnp.bfloat16)
    w2c = _padw(mlp2_w2 @ b3, rows=_HP2, cols=_HP2).astype(jnp.bfloat16)
    cv = _padv(mlp1_b2 @ a3 + mlp2_b2 @ b3 + mlp3_b1, cols=_HP2)
    v1 = jnp.pad(mlp2_w1, ((0, 0), (0, _HP2 - _HID)))
    w3 = _padv(mlp3_w2[:, 0], cols=_HP2)
    b3s = mlp3_b2.reshape(1, 1)

    eo = _tc_edge(pe, qe, edge_attr, v1, _padv(mlp2_b1, cols=_HP2),
                  w1c, w2c, cv, w3, b3s)
    return jnp.reshape(eo, (_E, 1))


# speedup vs baseline: 1.0261x; 1.0261x over previous
import functools

import jax
import jax.numpy as jnp
from jax import lax
from jax.experimental import pallas as pl
from jax.experimental.pallas import tpu as pltpu
from jax.experimental.pallas import tpu_sc as plsc

_N = 10000
_E = 320000
_DIN = 128
_HID = 72
_HP = 80
_NL = 3

_NC = 2
_NS = 16
_NW = _NC * _NS
_ET = _E // _NW
_CH = 80
_NCH = _ET // _CH
_RB = 1000
_NRB = _N // _RB
_HP2 = 128
_BE = 3200
_NBE = _E // _BE
_ZR = 200


def _sc_mesh():
    return plsc.VectorSubcoreMesh(core_axis_name="c", subcore_axis_name="s")


def _worker(base_len):
    cid = lax.axis_index("c")
    sid = lax.axis_index("s")
    wid = sid * _NC + cid
    return cid, sid, wid * base_len


def _sc_deg_body(col2_hbm, out0_hbm, out1_hbm, idx_c2, ones_v, zb_v, acc_sh, sem):
    cid, sid, _ = _worker(0)
    wid = sid * _NC + cid

    def fill_ones(k, carry):
        ones_v[pl.ds(k * 16, 16)] = jnp.full((16,), 1.0, jnp.float32)
        return carry
    lax.fori_loop(0, _CH // 16, fill_ones, 0)

    def fill_z(k, carry):
        zb_v[pl.ds(k * 16, 16)] = jnp.zeros((16,), jnp.float32)
        return carry
    lax.fori_loop(0, 1024 // 16, fill_z, 0)

    @pl.when(sid < _NRB)
    def _():
        pltpu.sync_copy(zb_v.at[pl.ds(0, _RB)], acc_sh.at[pl.ds(sid * _RB, _RB)])
    plsc.subcore_barrier()

    pltpu.sync_copy(col2_hbm.at[wid], idx_c2)

    def step(j, carry):
        pltpu.sync_copy(ones_v, acc_sh.at[idx_c2.at[j]], add=True)
        return carry
    lax.fori_loop(0, _NCH, step, 0)
    plsc.subcore_barrier()

    @pl.when(sid < _NRB)
    def _():
        pltpu.sync_copy(acc_sh.at[pl.ds(sid * _RB, _RB)], zb_v.at[pl.ds(0, _RB)])

    @pl.when((sid < _NRB) & (cid == 0))
    def _():
        pltpu.sync_copy(zb_v.at[pl.ds(0, _RB)], out0_hbm.at[pl.ds(sid * _RB, _RB)])

    @pl.when((sid < _NRB) & (cid == 1))
    def _():
        pltpu.sync_copy(zb_v.at[pl.ds(0, _RB)], out1_hbm.at[pl.ds(sid * _RB, _RB)])


def _sc_deg(col2):
    f = pl.kernel(
        _sc_deg_body,
        out_type=(jax.ShapeDtypeStruct((_N,), jnp.float32),
                  jax.ShapeDtypeStruct((_N,), jnp.float32)),
        mesh=_sc_mesh(),
        compiler_params=pltpu.CompilerParams(use_tc_tiling_on_sc=False),
        scratch_types=[
            pltpu.VMEM((_NCH, _CH), jnp.int32),
            pltpu.VMEM((_CH,), jnp.float32),
            pltpu.VMEM((1024,), jnp.float32),
            pltpu.VMEM_SHARED((_N,), jnp.float32),
            pltpu.SemaphoreType.DMA,
        ],
    )
    return f(col2)


def _sc_spmm_body(zt_hbm, row2_hbm, col2_hbm, out_hbm,
                  idx_r2, idx_c2, rows_a, rows_b, rows_c, zb_v, acc_sh,
                  sem_a, sem_b, sem_c, ssc_a, ssc_b, ssc_c):
    cid, sid, _ = _worker(0)
    wid = sid * _NC + cid

    def fill_z(k, carry):
        i = k // (_HP // 16)
        j = k % (_HP // 16)
        zb_v[i, pl.ds(j * 16, 16)] = jnp.zeros((16,), jnp.float32)
        return carry
    lax.fori_loop(0, _ZR * (_HP // 16), fill_z, 0)

    @pl.when(sid < _NRB)
    def _():
        def zcp(t, carry):
            pltpu.sync_copy(zb_v, acc_sh.at[pl.ds(sid * _RB + t * _ZR, _ZR)])
            return carry
        lax.fori_loop(0, _RB // _ZR, zcp, 0)
    plsc.subcore_barrier()

    pltpu.sync_copy(row2_hbm.at[wid], idx_r2)
    pltpu.sync_copy(col2_hbm.at[wid], idx_c2)

    def startg(j, rv, s):
        pltpu.async_copy(zt_hbm.at[idx_r2.at[j]], rv, s)

    def waitg(j, rv, s):
        pltpu.make_async_copy(zt_hbm.at[idx_r2.at[j]], rv, s).wait()

    def startsc(j, rv, s):
        pltpu.async_copy(rv, acc_sh.at[idx_c2.at[j]], s, add=True)

    def waitsc(j, rv, s):
        pltpu.make_async_copy(rv, acc_sh.at[idx_c2.at[j]], s).wait()

    startg(0, rows_a, sem_a)
    startg(1, rows_b, sem_b)
    startg(2, rows_c, sem_c)

    def step(t, carry):
        ja = 3 * t
        waitg(ja, rows_a, sem_a)
        startsc(ja, rows_a, ssc_a)
        waitg(ja + 1, rows_b, sem_b)
        startsc(ja + 1, rows_b, ssc_b)
        waitg(ja + 2, rows_c, sem_c)
        startsc(ja + 2, rows_c, ssc_c)
        waitsc(ja, rows_a, ssc_a)
        startg(ja + 3, rows_a, sem_a)
        waitsc(ja + 1, rows_b, ssc_b)
        startg(ja + 4, rows_b, sem_b)
        waitsc(ja + 2, rows_c, ssc_c)

        @pl.when(ja + 5 < _NCH)
        def _():
            startg(ja + 5, rows_c, sem_c)
        return carry
    lax.fori_loop(0, _NCH // 3, step, 0)
    jt = (_NCH // 3) * 3
    waitg(jt, rows_a, sem_a)
    startsc(jt, rows_a, ssc_a)
    waitg(jt + 1, rows_b, sem_b)
    startsc(jt + 1, rows_b, ssc_b)
    waitsc(jt, rows_a, ssc_a)
    waitsc(jt + 1, rows_b, ssc_b)
    plsc.subcore_barrier()

    @pl.when(sid < _NRB)
    def _():
        def wcp(t, carry):
            off = sid * _RB + t * _ZR
            pltpu.sync_copy(acc_sh.at[pl.ds(off, _ZR)], zb_v)
            pltpu.sync_copy(zb_v, out_hbm.at[cid, pl.ds(off, _ZR)])
            return carry
        lax.fori_loop(0, _RB // _ZR, wcp, 0)


def _sc_spmm(zt, row2, col2):
    f = pl.kernel(
        _sc_spmm_body,
        out_type=jax.ShapeDtypeStruct((_NC, _N, _HP), jnp.float32),
        mesh=_sc_mesh(),
        compiler_params=pltpu.CompilerParams(use_tc_tiling_on_sc=False),
        scratch_types=[
            pltpu.VMEM((_NCH, _CH), jnp.int32),
            pltpu.VMEM((_NCH, _CH), jnp.int32),
            pltpu.VMEM((_CH, _HP), jnp.float32),
            pltpu.VMEM((_CH, _HP), jnp.float32),
            pltpu.VMEM((_CH, _HP), jnp.float32),
            pltpu.VMEM((_ZR, _HP), jnp.float32),
            pltpu.VMEM_SHARED((_N, _HP), jnp.float32),
            pltpu.SemaphoreType.DMA,
            pltpu.SemaphoreType.DMA,
            pltpu.SemaphoreType.DMA,
            pltpu.SemaphoreType.DMA,
            pltpu.SemaphoreType.DMA,
            pltpu.SemaphoreType.DMA,
        ],
    )
    return f(zt, row2, col2)


def _sc_gather2_body(p_hbm, q_hbm, row2_hbm, col2_hbm, pe_hbm, qe_hbm,
                     idx_r2, idx_c2, pr_a, qr_a, pr_b, qr_b,
                     sp_a, sq_a, sp_b, sq_b):
    cid, sid, ebase = _worker(_ET)
    wid = sid * _NC + cid

    pltpu.sync_copy(row2_hbm.at[wid], idx_r2)
    pltpu.sync_copy(col2_hbm.at[wid], idx_c2)

    def start(j, pr, qr, sp, sq):
        pltpu.async_copy(p_hbm.at[idx_r2.at[j]], pr, sp)
        pltpu.async_copy(q_hbm.at[idx_c2.at[j]], qr, sq)

    def finish(j, pr, qr, sp, sq):
        pltpu.make_async_copy(p_hbm.at[idx_r2.at[j]], pr, sp).wait()
        pltpu.make_async_copy(q_hbm.at[idx_c2.at[j]], qr, sq).wait()
        off = ebase + j * _CH
        pltpu.sync_copy(pr, pe_hbm.at[pl.ds(off, _CH)])
        pltpu.sync_copy(qr, qe_hbm.at[pl.ds(off, _CH)])

    start(0, pr_a, qr_a, sp_a, sq_a)

    def step(t, carry):
        start(2 * t + 1, pr_b, qr_b, sp_b, sq_b)
        finish(2 * t, pr_a, qr_a, sp_a, sq_a)
        start(2 * t + 2, pr_a, qr_a, sp_a, sq_a)
        finish(2 * t + 1, pr_b, qr_b, sp_b, sq_b)
        return carry
    lax.fori_loop(0, (_NCH - 1) // 2, step, 0)
    finish(_NCH - 1, pr_a, qr_a, sp_a, sq_a)


def _sc_gather2(p, q, row2, col2):
    f = pl.kernel(
        _sc_gather2_body,
        out_type=(jax.ShapeDtypeStruct((_E, _HP2), jnp.float32),
                  jax.ShapeDtypeStruct((_E, _HP2), jnp.float32)),
        mesh=_sc_mesh(),
        compiler_params=pltpu.CompilerParams(use_tc_tiling_on_sc=False),
        scratch_types=[
            pltpu.VMEM((_NCH, _CH), jnp.int32),
            pltpu.VMEM((_NCH, _CH), jnp.int32),
            pltpu.VMEM((_CH, _HP2), jnp.float32),
            pltpu.VMEM((_CH, _HP2), jnp.float32),
            pltpu.VMEM((_CH, _HP2), jnp.float32),
            pltpu.VMEM((_CH, _HP2), jnp.float32),
            pltpu.SemaphoreType.DMA,
            pltpu.SemaphoreType.DMA,
            pltpu.SemaphoreType.DMA,
            pltpu.SemaphoreType.DMA,
        ],
    )
    return f(p, q, row2, col2)


def _dinv_of(deg):
    return jnp.where(deg > 0, lax.rsqrt(jnp.maximum(deg, 1e-12)), 0.0)


def _tc_node_body(x_ref, w_ref, b_ref, o_ref):
    o_ref[...] = jax.nn.relu(
        jnp.dot(x_ref[...], w_ref[...], preferred_element_type=jnp.float32)
        + b_ref[...])


def _tc_node(x, w, b):
    return pl.pallas_call(
        _tc_node_body,
        grid=(_NRB,),
        in_specs=[pl.BlockSpec((_RB, _DIN), lambda i: (i, 0)),
                  pl.BlockSpec((_DIN, _HP), lambda i: (0, 0)),
                  pl.BlockSpec((1, _HP), lambda i: (0, 0))],
        out_specs=pl.BlockSpec((_RB, _HP), lambda i: (i, 0)),
        out_shape=jax.ShapeDtypeStruct((_N, _HP), jnp.float32),
    )(x, w, b)


def _tc_pre_body(src_ref, k1_ref, k2_ref, deg_ref, w_ref, o_ref):
    h = src_ref[...] * k1_ref[...] + k2_ref[...]
    dinv = _dinv_of(deg_ref[...])
    o_ref[...] = jnp.dot(h, w_ref[...],
                         preferred_element_type=jnp.float32) * dinv


def _tc_pre(src, k1, k2, deg, w):
    return pl.pallas_call(
        _tc_pre_body,
        grid=(_NRB,),
        in_specs=[pl.BlockSpec((_RB, _HP), lambda i: (i, 0)),
                  pl.BlockSpec((1, _HP), lambda i: (0, 0)),
                  pl.BlockSpec((1, _HP), lambda i: (0, 0)),
                  pl.BlockSpec((_RB, 1), lambda i: (i, 0)),
                  pl.BlockSpec((_HP, _HP), lambda i: (0, 0))],
        out_specs=pl.BlockSpec((_RB, _HP), lambda i: (i, 0)),
        out_shape=jax.ShapeDtypeStruct((_N, _HP), jnp.float32),
    )(src, k1, k2, deg, w)


def _tc_post_body(p0_ref, p1_ref, src_ref, k1_ref, k2_ref, deg_ref, w_ref,
                  b_ref, o_ref, s1_ref, s2_ref):
    h = src_ref[...] * k1_ref[...] + k2_ref[...]
    dinv = _dinv_of(deg_ref[...])
    agg = (p0_ref[...] + p1_ref[...]) * dinv
    o = jax.nn.relu(agg
                    + jnp.dot(h, w_ref[...], preferred_element_type=jnp.float32)
                    + b_ref[...])
    o_ref[...] = o
    s1_ref[...] = jnp.sum(o, axis=0, keepdims=True)[None]
    s2_ref[...] = jnp.sum(o * o, axis=0, keepdims=True)[None]


def _tc_post(p0, p1, src, k1, k2, deg, w, b):
    return pl.pallas_call(
        _tc_post_body,
        grid=(_NRB,),
        in_specs=[pl.BlockSpec((_RB, _HP), lambda i: (i, 0)),
                  pl.BlockSpec((_RB, _HP), lambda i: (i, 0)),
                  pl.BlockSpec((_RB, _HP), lambda i: (i, 0)),
                  pl.BlockSpec((1, _HP), lambda i: (0, 0)),
                  pl.BlockSpec((1, _HP), lambda i: (0, 0)),
                  pl.BlockSpec((_RB, 1), lambda i: (i, 0)),
                  pl.BlockSpec((_HP, _HP), lambda i: (0, 0)),
                  pl.BlockSpec((1, _HP), lambda i: (0, 0))],
        out_specs=[pl.BlockSpec((_RB, _HP), lambda i: (i, 0)),
                   pl.BlockSpec((1, 1, _HP), lambda i: (i, 0, 0)),
                   pl.BlockSpec((1, 1, _HP), lambda i: (i, 0, 0))],
        out_shape=[jax.ShapeDtypeStruct((_N, _HP), jnp.float32),
                   jax.ShapeDtypeStruct((_NRB, 1, _HP), jnp.float32),
                   jax.ShapeDtypeStruct((_NRB, 1, _HP), jnp.float32)],
    )(p0, p1, src, k1, k2, deg, w, b)


def _tc_pq_body(src_ref, k1_ref, k2_ref, wa_ref, wb_ref, b1_ref,
                p_ref, q_ref):
    h = src_ref[...] * k1_ref[...] + k2_ref[...]
    p_ref[...] = jnp.dot(h, wa_ref[...],
                         preferred_element_type=jnp.float32) + b1_ref[...]
    q_ref[...] = jnp.dot(h, wb_ref[...], preferred_element_type=jnp.float32)


def _tc_pq(src, k1, k2, wa, wb, b1):
    return pl.pallas_call(
        _tc_pq_body,
        grid=(_NRB,),
        in_specs=[pl.BlockSpec((_RB, _HP), lambda i: (i, 0)),
                  pl.BlockSpec((1, _HP), lambda i: (0, 0)),
                  pl.BlockSpec((1, _HP), lambda i: (0, 0)),
                  pl.BlockSpec((_HP, _HP2), lambda i: (0, 0)),
                  pl.BlockSpec((_HP, _HP2), lambda i: (0, 0)),
                  pl.BlockSpec((1, _HP2), lambda i: (0, 0))],
        out_specs=[pl.BlockSpec((_RB, _HP2), lambda i: (i, 0)),
                   pl.BlockSpec((_RB, _HP2), lambda i: (i, 0))],
        out_shape=[jax.ShapeDtypeStruct((_N, _HP2), jnp.float32),
                   jax.ShapeDtypeStruct((_N, _HP2), jnp.float32)],
    )(src, k1, k2, wa, wb, b1)


def _tc_edge_body(pe_ref, qe_ref, ea_ref, v1_ref, b21_ref, w1c_ref, w2c_ref,
                  cv_ref, w3_ref, b3_ref, o_ref):
    t1 = jnp.tanh(pe_ref[...] + qe_ref[...])
    t2 = jnp.tanh(jnp.dot(ea_ref[...], v1_ref[...],
                          preferred_element_type=jnp.float32) + b21_ref[...])
    s = jnp.tanh(jnp.dot(t1, w1c_ref[...], preferred_element_type=jnp.float32)
                 + jnp.dot(t2, w2c_ref[...], preferred_element_type=jnp.float32)
                 + cv_ref[...])
    o = lax.dot_general(w3_ref[...], s, (((1,), (1,)), ((), ())),
                        preferred_element_type=jnp.float32) + b3_ref[...]
    o_ref[...] = o[None]


def _tc_edge(pe, qe, ea, v1, b21, w1c, w2c, cv, w3, b3):
    return pl.pallas_call(
        _tc_edge_body,
        grid=(_NBE,),
        in_specs=[pl.BlockSpec((_BE, _HP2), lambda i: (i, 0)),
                  pl.BlockSpec((_BE, _HP2), lambda i: (i, 0)),
                  pl.BlockSpec((_BE, 16), lambda i: (i, 0)),
                  pl.BlockSpec((16, _HP2), lambda i: (0, 0)),
                  pl.BlockSpec((1, _HP2), lambda i: (0, 0)),
                  pl.BlockSpec((_HP2, _HP2), lambda i: (0, 0)),
                  pl.BlockSpec((_HP2, _HP2), lambda i: (0, 0)),
                  pl.BlockSpec((1, _HP2), lambda i: (0, 0)),
                  pl.BlockSpec((1, _HP2), lambda i: (0, 0)),
                  pl.BlockSpec((1, 1), lambda i: (0, 0))],
        out_specs=pl.BlockSpec((1, 1, _BE), lambda i: (i, 0, 0)),
        out_shape=jax.ShapeDtypeStruct((_NBE, 1, _BE), jnp.float32),
    )(pe, qe, ea, v1, b21, w1c, w2c, cv, w3, b3)


def _padw(w, rows=_HP, cols=_HP):
    r, c = w.shape
    return jnp.pad(w, ((0, (rows - r) if r == _HID else 0), (0, cols - c)))


def _padv(v, cols=_HP):
    return jnp.pad(v.reshape(1, -1), ((0, 0), (0, cols - v.shape[-1])))


def kernel(x, edge_index, edge_attr, node_w, node_b, init_w, root_w, arma_b,
           bn_gamma, bn_beta, mlp1_w1, mlp1_b1, mlp1_w2, mlp1_b2,
           mlp2_w1, mlp2_b1, mlp2_w2, mlp2_b2,
           mlp3_w1, mlp3_b1, mlp3_w2, mlp3_b2):
    row2 = edge_index[0].reshape(_NW, _NCH, _CH)
    col2 = edge_index[1].reshape(_NW, _NCH, _CH)

    degp0, degp1 = _sc_deg(col2)
    deg = (degp0 + degp1).reshape(_N, 1)

    h = _tc_node(x, jnp.pad(node_w, ((0, 0), (0, _HP - _HID))), _padv(node_b))

    ones = jnp.ones((1, _HP), jnp.float32)
    zeros = jnp.zeros((1, _HP), jnp.float32)
    k1, k2, src = ones, zeros, h
    for l in range(_NL):
        zt = _tc_pre(src, k1, k2, deg, _padw(init_w[l]))
        part = _sc_spmm(zt, row2, col2)
        out, s1, s2 = _tc_post(part[0], part[1], src, k1, k2, deg,
                               _padw(root_w[l]), _padv(arma_b[l]))
        mean = jnp.sum(s1, axis=(0, 1)) * (1.0 / _N)
        var = jnp.sum(s2, axis=(0, 1)) * (1.0 / _N) - mean * mean
        g = _padv(bn_gamma[l])[0]
        k1v = g * lax.rsqrt(var + 1e-5)
        k2v = _padv(bn_beta[l])[0] - mean * k1v
        k1, k2, src = k1v.reshape(1, _HP), k2v.reshape(1, _HP), out

    wa = _padw(mlp1_w1[:_HID], cols=_HP2)
    wb = _padw(mlp1_w1[_HID:], cols=_HP2)
    p, q = _tc_pq(src, k1, k2, wa, wb, _padv(mlp1_b1, cols=_HP2))

    pe, qe = _sc_gather2(p, q, row2, col2)

    a3 = mlp3_w1[:_HID]
    b3 = mlp3_w1[_HID:]
    w1c = _padw(mlp1_w2 @ a3, rows=_HP2, cols=_HP2)
    w2c = _padw(mlp2_w2 @ b3, rows=_HP2, cols=_HP2)
    cv = _padv(mlp1_b2 @ a3 + mlp2_b2 @ b3 + mlp3_b1, cols=_HP2)
    v1 = jnp.pad(mlp2_w1, ((0, 0), (0, _HP2 - _HID)))
    w3 = _padv(mlp3_w2[:, 0], cols=_HP2)
    b3s = mlp3_b2.reshape(1, 1)

    eo = _tc_edge(pe, qe, edge_attr, v1, _padv(mlp2_b1, cols=_HP2),
                  w1c, w2c, cv, w3, b3s)
    return jnp.reshape(eo, (_E, 1))
